# baseline (speedup 1.0000x reference)
import functools

import jax
import jax.numpy as jnp
from jax import lax
from jax.experimental import pallas as pl
from jax.experimental.pallas import tpu as pltpu
from jax.experimental.pallas import tpu_sc as plsc

USER_NUM = 1000000
ITEM_NUM = 1000000
FACTOR = 16
BATCH = 16384

L = 16
NW = 32
BPW = BATCH // NW
NCHUNK = 4
CHUNK = BPW // NCHUNK

CHW = 512
NFULL = USER_NUM // CHW
CH_BIG = NFULL - 61 * NW
LASTC0 = (NFULL - 1) * CHW
HCAP = 1024
HSLACK = HCAP + L
SENT = BATCH + 64
INVN = BATCH + BPW


def _p1_body(user_h, itemi_h, itemj_h, eu_t, ei_t,
             rows_u_o, rows_i_o, rows_j_o, inv_u_o, inv_i_o, inv_j_o,
             slab_u, slab_v, slab_t, slab_t2, idxbuf,
             hidx_u, hpos_u, hidx_i, hpos_i, hidx_j, hpos_j,
             rows_u, rows_i, rows_j,
             slots_u, posb_u, slots_i, posb_i, slots_j, posb_j,
             cidx, cpos, sem):
    wid = lax.axis_index("s") * 2 + lax.axis_index("c")
    start = jnp.where(wid < CH_BIG, 62 * wid, 61 * wid + CH_BIG)
    rlo = start * CHW
    rhi = jnp.where(wid == NW - 1, USER_NUM, (start + 61) * CHW
                    + jnp.where(wid < CH_BIG, CHW, 0))

    iota = lax.iota(jnp.int32, L)

    sentv = jnp.full((L,), SENT, jnp.int32)
    for posb in (posb_u, posb_i, posb_j):
        for g in range(8):
            for v in range(128 // L):
                plsc.store_scatter(
                    posb, [jnp.full((L,), g, jnp.int32), v * L + iota], sentv)

    def bin_set(idx_h, hidx, hpos):
        cur = jnp.int32(0)
        for p in range(BATCH // 2048):
            pltpu.sync_copy(idx_h.at[pl.ds(p * 2048, 2048)], idxbuf)

            def scan(i, c):
                x = idxbuf[pl.ds(i * L, L)]
                m = (x >= rlo) & (x < rhi)
                n = plsc.all_reduce_population_count(m)[0]

                @pl.when(n > 0)
                def _():
                    pos = p * 2048 + i * L + iota
                    plsc.store_compressed(hidx.at[pl.ds(c, L)], x, mask=m)
                    plsc.store_compressed(hpos.at[pl.ds(c, L)], pos, mask=m)

                return c + n

            cur = lax.fori_loop(0, 2048 // L, scan, cur)
        return cur

    nu = bin_set(user_h, hidx_u, hpos_u)
    ni = bin_set(itemi_h, hidx_i, hpos_i)
    nj = bin_set(itemj_h, hidx_j, hpos_j)

    def serve(slab, hidx, hpos, nh, rows, slots, posb, clo, chi, c0, gc):
        def cscan(v, cc):
            x = hidx[pl.ds(v * L, L)]
            pp = hpos[pl.ds(v * L, L)]
            m = (x >= clo) & (x < chi)
            n = plsc.all_reduce_population_count(m)[0]

            @pl.when(n > 0)
            def _():
                plsc.store_compressed(cidx.at[pl.ds(cc, L)], x - c0, mask=m)
                plsc.store_compressed(cpos.at[pl.ds(cc, L)], pp, mask=m)

            return cc + n

        ccnt = lax.fori_loop(0, (nh + L - 1) // L, cscan, jnp.int32(0))

        def emitg(gi, carry):
            lane = gi * L
            valid = (lane + iota) < ccnt
            cols = jnp.where(valid, cidx[pl.ds(lane, L)], 0)
            poss = cpos[pl.ds(lane, L)]
            slotv = gc + lane + iota
            for f in range(FACTOR):
                fv = jnp.full((L,), f, jnp.int32)
                vals = plsc.load_gather(slab, [fv, cols])
                plsc.store_scatter(rows, [slotv * L + f], vals, mask=valid)
            rowp = lax.shift_right_logical(slotv, 7)
            colp = slotv & 127
            plsc.store_scatter(posb, [rowp, colp], poss, mask=valid)
            plsc.store_scatter(slots, [rowp, colp], wid * HCAP + slotv,
                               mask=valid)
            return carry

        lax.fori_loop(0, (ccnt + L - 1) // L, emitg, jnp.int32(0))
        return gc + ccnt

    def chunk_step(k, carry):
        gu, gi, gj = carry
        ch = start + k
        clo = ch * CHW
        chi = jnp.minimum(clo + CHW, NFULL * CHW)
        c0 = pl.multiple_of(jnp.minimum(clo, LASTC0), CHW)
        cu = pltpu.async_copy(eu_t.at[:, pl.ds(c0, CHW)], slab_u, sem)
        cv = pltpu.async_copy(ei_t.at[:, pl.ds(c0, CHW)], slab_v, sem)
        cu.wait()
        cv.wait()
        gu2 = serve(slab_u, hidx_u, hpos_u, nu, rows_u, slots_u, posb_u,
                    clo, chi, c0, gu)
        gi2 = serve(slab_v, hidx_i, hpos_i, ni, rows_i, slots_i, posb_i,
                    clo, chi, c0, gi)
        gj2 = serve(slab_v, hidx_j, hpos_j, nj, rows_j, slots_j, posb_j,
                    clo, chi, c0, gj)
        return gu2, gi2, gj2

    gu, gi, gj = lax.fori_loop(
        0, 62, chunk_step, (jnp.int32(0), jnp.int32(0), jnp.int32(0)))

    @pl.when(wid == NW - 1)
    def _():
        tl = jnp.int32(NFULL * CHW)
        cu = pltpu.async_copy(eu_t.at[:, pl.ds(NFULL * CHW, 64)], slab_t, sem)
        cv = pltpu.async_copy(ei_t.at[:, pl.ds(NFULL * CHW, 64)], slab_t2, sem)
        cu.wait()
        cv.wait()
        serve(slab_t, hidx_u, hpos_u, nu, rows_u, slots_u, posb_u,
              tl, jnp.int32(USER_NUM), tl, gu)
        serve(slab_t2, hidx_i, hpos_i, ni, rows_i, slots_i, posb_i,
              tl, jnp.int32(USER_NUM), tl, gi)
        serve(slab_t2, hidx_j, hpos_j, nj, rows_j, slots_j, posb_j,
              tl, jnp.int32(USER_NUM), tl, gj)

    base = wid * HCAP * L
    pltpu.sync_copy(rows_u, rows_u_o.at[pl.ds(base, HCAP * L)])
    pltpu.sync_copy(rows_i, rows_i_o.at[pl.ds(base, HCAP * L)])
    pltpu.sync_copy(rows_j, rows_j_o.at[pl.ds(base, HCAP * L)])
    for inv_o, slots, posb in ((inv_u_o, slots_u, posb_u),
                               (inv_i_o, slots_i, posb_i),
                               (inv_j_o, slots_j, posb_j)):
        for g in range(8):
            pltpu.async_copy(slots.at[g], inv_o.at[posb.at[g]], sem).wait()


def _p2_body(inv_u_h, inv_i_h, inv_j_h, ru_h, ri_h, rj_h,
             item_i_h, item_j_h, bias_h,
             out_pi, out_pj, out_pw,
             inv_u, inv_i, inv_j, idx_i, idx_j,
             rows_u, rows_i, rows_j, bias_i_v, bias_j_v,
             pw_v, pred_i_v, pred_j_v, sem):
    wid = lax.axis_index("s") * 2 + lax.axis_index("c")
    base = wid * BPW

    pltpu.sync_copy(inv_u_h.at[wid], inv_u)
    pltpu.sync_copy(inv_i_h.at[wid], inv_i)
    pltpu.sync_copy(inv_j_h.at[wid], inv_j)
    pltpu.sync_copy(item_i_h.at[wid], idx_i)
    pltpu.sync_copy(item_j_h.at[wid], idx_j)

    copies = []
    for c in range(NCHUNK):
        dst = pl.ds(c * CHUNK, CHUNK)
        copies.append(pltpu.async_copy(ru_h.at[inv_u.at[c]], rows_u.at[dst], sem))
        copies.append(pltpu.async_copy(ri_h.at[inv_i.at[c]], rows_i.at[dst], sem))
        copies.append(pltpu.async_copy(rj_h.at[inv_j.at[c]], rows_j.at[dst], sem))
        copies.append(pltpu.async_copy(bias_h.at[idx_i.at[c]], bias_i_v.at[dst], sem))
        copies.append(pltpu.async_copy(bias_h.at[idx_j.at[c]], bias_j_v.at[dst], sem))
    for cp in copies:
        cp.wait()

    iota = lax.iota(jnp.int32, L)

    def block(b, carry):
        rbase = b * L
        ridx = rbase + iota
        acc_i = bias_i_v[pl.ds(rbase, L)]
        acc_j = bias_j_v[pl.ds(rbase, L)]
        for f in range(FACTOR):
            cf = jnp.full((L,), f, jnp.int32)
            uc = plsc.load_gather(rows_u, [ridx, cf])
            ic = plsc.load_gather(rows_i, [ridx, cf])
            jc = plsc.load_gather(rows_j, [ridx, cf])
            pwc = uc * ic
            plsc.store_scatter(pw_v, [cf, ridx], pwc)
            acc_i = acc_i + pwc
            acc_j = acc_j + uc * jc
        pred_i_v[pl.ds(rbase, L)] = acc_i
        pred_j_v[pl.ds(rbase, L)] = acc_j
        return carry

    lax.fori_loop(0, BPW // L, block, 0)

    pltpu.sync_copy(pred_i_v, out_pi.at[wid])
    pltpu.sync_copy(pred_j_v, out_pj.at[wid])
    pltpu.sync_copy(pw_v, out_pw.at[:, pl.ds(base, BPW)])


@jax.jit
def _bpr_sc(user, item_i, item_j, eu_t, ei_t, bias1d, ii3, ij3):
    mesh = plsc.VectorSubcoreMesh(core_axis_name="c", subcore_axis_name="s")
    p1 = functools.partial(
        pl.kernel,
        mesh=mesh,
        compiler_params=pltpu.CompilerParams(
            needs_layout_passes=False, use_tc_tiling_on_sc=True),
        out_type=[
            jax.ShapeDtypeStruct((NW * HCAP * L,), jnp.float32),
            jax.ShapeDtypeStruct((NW * HCAP * L,), jnp.float32),
            jax.ShapeDtypeStruct((NW * HCAP * L,), jnp.float32),
            jax.ShapeDtypeStruct((INVN,), jnp.int32),
            jax.ShapeDtypeStruct((INVN,), jnp.int32),
            jax.ShapeDtypeStruct((INVN,), jnp.int32),
        ],
        scratch_types=[
            pltpu.VMEM((FACTOR, CHW), jnp.float32),
            pltpu.VMEM((FACTOR, CHW), jnp.float32),
            pltpu.VMEM((FACTOR, 64), jnp.float32),
            pltpu.VMEM((FACTOR, 64), jnp.float32),
            pltpu.VMEM((2048,), jnp.int32),
            pltpu.VMEM((HSLACK,), jnp.int32),
            pltpu.VMEM((HSLACK,), jnp.int32),
            pltpu.VMEM((HSLACK,), jnp.int32),
            pltpu.VMEM((HSLACK,), jnp.int32),
            pltpu.VMEM((HSLACK,), jnp.int32),
            pltpu.VMEM((HSLACK,), jnp.int32),
            pltpu.VMEM((HCAP * L,), jnp.float32),
            pltpu.VMEM((HCAP * L,), jnp.float32),
            pltpu.VMEM((HCAP * L,), jnp.float32),
            pltpu.VMEM((8, 128), jnp.int32),
            pltpu.VMEM((8, 128), jnp.int32),
            pltpu.VMEM((8, 128), jnp.int32),
            pltpu.VMEM((8, 128), jnp.int32),
            pltpu.VMEM((8, 128), jnp.int32),
            pltpu.VMEM((8, 128), jnp.int32),
            pltpu.VMEM((112,), jnp.int32),
            pltpu.VMEM((112,), jnp.int32),
            pltpu.SemaphoreType.DMA,
        ],
    )(_p1_body)
    ru, ri, rj, ivu, ivi, ivj = p1(user, item_i, item_j, eu_t, ei_t)

    p2 = functools.partial(
        pl.kernel,
        mesh=mesh,
        compiler_params=pltpu.CompilerParams(
            needs_layout_passes=False, use_tc_tiling_on_sc=False),
        out_type=[
            jax.ShapeDtypeStruct((NW, BPW), jnp.float32),
            jax.ShapeDtypeStruct((NW, BPW), jnp.float32),
            jax.ShapeDtypeStruct((FACTOR, BATCH), jnp.float32),
        ],
        scratch_types=[
            pltpu.VMEM((NCHUNK, CHUNK), jnp.int32),
            pltpu.VMEM((NCHUNK, CHUNK), jnp.int32),
            pltpu.VMEM((NCHUNK, CHUNK), jnp.int32),
            pltpu.VMEM((NCHUNK, CHUNK), jnp.int32),
            pltpu.VMEM((NCHUNK, CHUNK), jnp.int32),
            pltpu.VMEM((BPW, FACTOR), jnp.float32),
            pltpu.VMEM((BPW, FACTOR), jnp.float32),
            pltpu.VMEM((BPW, FACTOR), jnp.float32),
            pltpu.VMEM((BPW,), jnp.float32),
            pltpu.VMEM((BPW,), jnp.float32),
            pltpu.VMEM((FACTOR, BPW), jnp.float32),
            pltpu.VMEM((BPW,), jnp.float32),
            pltpu.VMEM((BPW,), jnp.float32),
            pltpu.SemaphoreType.DMA,
        ],
    )(_p2_body)
    pi, pj, pw = p2(
        ivu.reshape(INVN // BPW, NCHUNK, CHUNK),
        ivi.reshape(INVN // BPW, NCHUNK, CHUNK),
        ivj.reshape(INVN // BPW, NCHUNK, CHUNK),
        ru.reshape(NW * HCAP, L), ri.reshape(NW * HCAP, L),
        rj.reshape(NW * HCAP, L),
        ii3, ij3, bias1d)
    return pi, pj, pw


def kernel(user, item_i, item_j, embed_user, embed_item, item_biases):
    u = user.astype(jnp.int32)
    ii = item_i.astype(jnp.int32)
    ij = item_j.astype(jnp.int32)
    pi, pj, pw_t = _bpr_sc(
        u, ii, ij, embed_user.T, embed_item.T,
        item_biases.T.reshape(ITEM_NUM),
        ii.reshape(NW, NCHUNK, CHUNK), ij.reshape(NW, NCHUNK, CHUNK))
    return pi.reshape(BATCH), pj.reshape(BATCH), pw_t.T


# speedup vs baseline: 1.0971x; 1.0971x over previous
import functools

import jax
import jax.numpy as jnp
from jax import lax
from jax.experimental import pallas as pl
from jax.experimental.pallas import tpu as pltpu
from jax.experimental.pallas import tpu_sc as plsc

USER_NUM = 1000000
ITEM_NUM = 1000000
FACTOR = 16
BATCH = 16384

L = 16
NW = 32
BPW = BATCH // NW
NCHUNK = 4
CHUNK = BPW // NCHUNK

CHW = 512
NFULL = USER_NUM // CHW
CH_BIG = NFULL - 61 * NW
LASTC0 = (NFULL - 1) * CHW
HCAP = 1024
HSLACK = HCAP + L
SENT = BATCH + 64
INVN = BATCH + BPW


def _p1_body(user_h, itemi_h, itemj_h, eu_t, ei_t,
             rows_u_o, rows_i_o, rows_j_o, inv_u_o, inv_i_o, inv_j_o,
             slab_u, slab_v, slab_t, slab_t2, idxbuf,
             hidx_u, hpos_u, hidx_i, hpos_i, hidx_j, hpos_j,
             rows_u, rows_i, rows_j,
             slots_u, posb_u, slots_i, posb_i, slots_j, posb_j,
             cidx, cpos, sem):
    wid = lax.axis_index("s") * 2 + lax.axis_index("c")
    start = jnp.where(wid < CH_BIG, 62 * wid, 61 * wid + CH_BIG)
    rlo = start * CHW
    rhi = jnp.where(wid == NW - 1, USER_NUM, (start + 61) * CHW
                    + jnp.where(wid < CH_BIG, CHW, 0))

    iota = lax.iota(jnp.int32, L)

    sentv = jnp.full((L,), SENT, jnp.int32)
    for posb in (posb_u, posb_i, posb_j):
        for g in range(8):
            for v in range(128 // L):
                plsc.store_scatter(
                    posb, [jnp.full((L,), g, jnp.int32), v * L + iota], sentv)

    def bin_set(idx_h, hidx, hpos):
        cur = jnp.int32(0)
        for p in range(BATCH // 2048):
            pltpu.sync_copy(idx_h.at[pl.ds(p * 2048, 2048)], idxbuf)

            def scan(i, c):
                x = idxbuf[pl.ds(i * L, L)]
                m = (x >= rlo) & (x < rhi)
                n = plsc.all_reduce_population_count(m)[0]

                @pl.when(n > 0)
                def _():
                    pos = p * 2048 + i * L + iota
                    plsc.store_compressed(hidx.at[pl.ds(c, L)], x, mask=m)
                    plsc.store_compressed(hpos.at[pl.ds(c, L)], pos, mask=m)

                return c + n

            cur = lax.fori_loop(0, 2048 // L, scan, cur)
        return cur

    nu = bin_set(user_h, hidx_u, hpos_u)
    ni = bin_set(itemi_h, hidx_i, hpos_i)
    nj = bin_set(itemj_h, hidx_j, hpos_j)

    def serve(slab, hidx, hpos, nh, rows, slots, posb, clo, chi, c0, gc):
        def cscan(v, cc):
            x = hidx[pl.ds(v * L, L)]
            pp = hpos[pl.ds(v * L, L)]
            m = (x >= clo) & (x < chi) & ((v * L + iota) < nh)
            n = plsc.all_reduce_population_count(m)[0]

            @pl.when(n > 0)
            def _():
                plsc.store_compressed(cidx.at[pl.ds(cc, L)], x - c0, mask=m)
                plsc.store_compressed(cpos.at[pl.ds(cc, L)], pp, mask=m)

            return cc + n

        ccnt = lax.fori_loop(0, (nh + L - 1) // L, cscan, jnp.int32(0))

        def emitg(gi, carry):
            lane = gi * L
            valid = (lane + iota) < ccnt
            cols = jnp.where(valid, cidx[pl.ds(lane, L)], 0)
            poss = cpos[pl.ds(lane, L)]
            slotv = gc + lane + iota
            for f in range(FACTOR):
                hv = jnp.full((L,), f // 8, jnp.int32)
                fv = jnp.full((L,), f % 8, jnp.int32)
                vals = plsc.load_gather(slab, [hv, fv, cols])
                plsc.store_scatter(rows, [slotv * L + f], vals, mask=valid)
            rowp = lax.shift_right_logical(slotv, 7)
            colp = slotv & 127
            plsc.store_scatter(posb, [rowp, colp], poss, mask=valid)
            plsc.store_scatter(slots, [rowp, colp], wid * HCAP + slotv,
                               mask=valid)
            return carry

        lax.fori_loop(0, (ccnt + L - 1) // L, emitg, jnp.int32(0))
        return gc + ccnt

    def chunk_step(k, carry):
        gu, gi, gj = carry
        ch = start + k
        clo = ch * CHW
        chi = jnp.minimum(clo + CHW, NFULL * CHW)
        c0 = pl.multiple_of(jnp.minimum(clo, LASTC0), CHW)
        cps = []
        for h in range(2):
            cps.append(pltpu.async_copy(
                eu_t.at[h, :, pl.ds(c0, CHW)], slab_u.at[h], sem))
            cps.append(pltpu.async_copy(
                ei_t.at[h, :, pl.ds(c0, CHW)], slab_v.at[h], sem))
        for cp in cps:
            cp.wait()
        gu2 = serve(slab_u, hidx_u, hpos_u, nu, rows_u, slots_u, posb_u,
                    clo, chi, c0, gu)
        gi2 = serve(slab_v, hidx_i, hpos_i, ni, rows_i, slots_i, posb_i,
                    clo, chi, c0, gi)
        gj2 = serve(slab_v, hidx_j, hpos_j, nj, rows_j, slots_j, posb_j,
                    clo, chi, c0, gj)
        return gu2, gi2, gj2

    gu, gi, gj = lax.fori_loop(
        0, 62, chunk_step, (jnp.int32(0), jnp.int32(0), jnp.int32(0)))

    @pl.when(wid == NW - 1)
    def _():
        tl = jnp.int32(NFULL * CHW)
        cps = []
        for h in range(2):
            cps.append(pltpu.async_copy(
                eu_t.at[h, :, pl.ds(NFULL * CHW, 64)], slab_t.at[h], sem))
            cps.append(pltpu.async_copy(
                ei_t.at[h, :, pl.ds(NFULL * CHW, 64)], slab_t2.at[h], sem))
        for cp in cps:
            cp.wait()
        serve(slab_t, hidx_u, hpos_u, nu, rows_u, slots_u, posb_u,
              tl, jnp.int32(USER_NUM), tl, gu)
        serve(slab_t2, hidx_i, hpos_i, ni, rows_i, slots_i, posb_i,
              tl, jnp.int32(USER_NUM), tl, gi)
        serve(slab_t2, hidx_j, hpos_j, nj, rows_j, slots_j, posb_j,
              tl, jnp.int32(USER_NUM), tl, gj)

    base = wid * HCAP * L
    pltpu.sync_copy(rows_u, rows_u_o.at[pl.ds(base, HCAP * L)])
    pltpu.sync_copy(rows_i, rows_i_o.at[pl.ds(base, HCAP * L)])
    pltpu.sync_copy(rows_j, rows_j_o.at[pl.ds(base, HCAP * L)])
    for inv_o, slots, posb in ((inv_u_o, slots_u, posb_u),
                               (inv_i_o, slots_i, posb_i),
                               (inv_j_o, slots_j, posb_j)):
        for g in range(8):
            pltpu.async_copy(slots.at[g], inv_o.at[posb.at[g]], sem).wait()


def _p2_body(inv_u_h, inv_i_h, inv_j_h, ru_h, ri_h, rj_h,
             item_i_h, item_j_h, bias_h,
             out_pi, out_pj, out_pw,
             inv_u, inv_i, inv_j, idx_i, idx_j,
             rows_u, rows_i, rows_j, bias_i_v, bias_j_v,
             pw_v, pred_i_v, pred_j_v, sem):
    wid = lax.axis_index("s") * 2 + lax.axis_index("c")
    base = wid * BPW

    pltpu.sync_copy(inv_u_h.at[wid], inv_u)
    pltpu.sync_copy(inv_i_h.at[wid], inv_i)
    pltpu.sync_copy(inv_j_h.at[wid], inv_j)
    pltpu.sync_copy(item_i_h.at[wid], idx_i)
    pltpu.sync_copy(item_j_h.at[wid], idx_j)

    copies = []
    for c in range(NCHUNK):
        dst = pl.ds(c * CHUNK, CHUNK)
        copies.append(pltpu.async_copy(ru_h.at[inv_u.at[c]], rows_u.at[dst], sem))
        copies.append(pltpu.async_copy(ri_h.at[inv_i.at[c]], rows_i.at[dst], sem))
        copies.append(pltpu.async_copy(rj_h.at[inv_j.at[c]], rows_j.at[dst], sem))
        copies.append(pltpu.async_copy(bias_h.at[idx_i.at[c]], bias_i_v.at[dst], sem))
        copies.append(pltpu.async_copy(bias_h.at[idx_j.at[c]], bias_j_v.at[dst], sem))
    for cp in copies:
        cp.wait()

    iota = lax.iota(jnp.int32, L)

    def block(b, carry):
        rbase = b * L
        ridx = rbase + iota
        acc_i = bias_i_v[pl.ds(rbase, L)]
        acc_j = bias_j_v[pl.ds(rbase, L)]
        for f in range(FACTOR):
            cf = jnp.full((L,), f, jnp.int32)
            uc = plsc.load_gather(rows_u, [ridx, cf])
            ic = plsc.load_gather(rows_i, [ridx, cf])
            jc = plsc.load_gather(rows_j, [ridx, cf])
            pwc = uc * ic
            plsc.store_scatter(pw_v, [cf, ridx], pwc)
            acc_i = acc_i + pwc
            acc_j = acc_j + uc * jc
        pred_i_v[pl.ds(rbase, L)] = acc_i
        pred_j_v[pl.ds(rbase, L)] = acc_j
        return carry

    lax.fori_loop(0, BPW // L, block, 0)

    pltpu.sync_copy(pred_i_v, out_pi.at[wid])
    pltpu.sync_copy(pred_j_v, out_pj.at[wid])
    pltpu.sync_copy(pw_v, out_pw.at[:, pl.ds(base, BPW)])


@jax.jit
def _bpr_sc(user, item_i, item_j, eu_t, ei_t, bias1d, ii3, ij3):
    mesh = plsc.VectorSubcoreMesh(core_axis_name="c", subcore_axis_name="s")
    p1 = functools.partial(
        pl.kernel,
        mesh=mesh,
        compiler_params=pltpu.CompilerParams(
            needs_layout_passes=False, use_tc_tiling_on_sc=True),
        out_type=[
            jax.ShapeDtypeStruct((NW * HCAP * L,), jnp.float32),
            jax.ShapeDtypeStruct((NW * HCAP * L,), jnp.float32),
            jax.ShapeDtypeStruct((NW * HCAP * L,), jnp.float32),
            jax.ShapeDtypeStruct((INVN,), jnp.int32),
            jax.ShapeDtypeStruct((INVN,), jnp.int32),
            jax.ShapeDtypeStruct((INVN,), jnp.int32),
        ],
        scratch_types=[
            pltpu.VMEM((2, 8, CHW), jnp.float32),
            pltpu.VMEM((2, 8, CHW), jnp.float32),
            pltpu.VMEM((2, 8, 64), jnp.float32),
            pltpu.VMEM((2, 8, 64), jnp.float32),
            pltpu.VMEM((2048,), jnp.int32),
            pltpu.VMEM((HSLACK,), jnp.int32),
            pltpu.VMEM((HSLACK,), jnp.int32),
            pltpu.VMEM((HSLACK,), jnp.int32),
            pltpu.VMEM((HSLACK,), jnp.int32),
            pltpu.VMEM((HSLACK,), jnp.int32),
            pltpu.VMEM((HSLACK,), jnp.int32),
            pltpu.VMEM((HCAP * L,), jnp.float32),
            pltpu.VMEM((HCAP * L,), jnp.float32),
            pltpu.VMEM((HCAP * L,), jnp.float32),
            pltpu.VMEM((8, 128), jnp.int32),
            pltpu.VMEM((8, 128), jnp.int32),
            pltpu.VMEM((8, 128), jnp.int32),
            pltpu.VMEM((8, 128), jnp.int32),
            pltpu.VMEM((8, 128), jnp.int32),
            pltpu.VMEM((8, 128), jnp.int32),
            pltpu.VMEM((112,), jnp.int32),
            pltpu.VMEM((112,), jnp.int32),
            pltpu.SemaphoreType.DMA,
        ],
    )(_p1_body)
    ru, ri, rj, ivu, ivi, ivj = p1(user, item_i, item_j, eu_t, ei_t)

    p2 = functools.partial(
        pl.kernel,
        mesh=mesh,
        compiler_params=pltpu.CompilerParams(
            needs_layout_passes=False, use_tc_tiling_on_sc=False),
        out_type=[
            jax.ShapeDtypeStruct((NW, BPW), jnp.float32),
            jax.ShapeDtypeStruct((NW, BPW), jnp.float32),
            jax.ShapeDtypeStruct((FACTOR, BATCH), jnp.float32),
        ],
        scratch_types=[
            pltpu.VMEM((NCHUNK, CHUNK), jnp.int32),
            pltpu.VMEM((NCHUNK, CHUNK), jnp.int32),
            pltpu.VMEM((NCHUNK, CHUNK), jnp.int32),
            pltpu.VMEM((NCHUNK, CHUNK), jnp.int32),
            pltpu.VMEM((NCHUNK, CHUNK), jnp.int32),
            pltpu.VMEM((BPW, FACTOR), jnp.float32),
            pltpu.VMEM((BPW, FACTOR), jnp.float32),
            pltpu.VMEM((BPW, FACTOR), jnp.float32),
            pltpu.VMEM((BPW,), jnp.float32),
            pltpu.VMEM((BPW,), jnp.float32),
            pltpu.VMEM((FACTOR, BPW), jnp.float32),
            pltpu.VMEM((BPW,), jnp.float32),
            pltpu.VMEM((BPW,), jnp.float32),
            pltpu.SemaphoreType.DMA,
        ],
    )(_p2_body)
    pi, pj, pw = p2(
        ivu.reshape(INVN // BPW, NCHUNK, CHUNK),
        ivi.reshape(INVN // BPW, NCHUNK, CHUNK),
        ivj.reshape(INVN // BPW, NCHUNK, CHUNK),
        ru.reshape(NW * HCAP, L), ri.reshape(NW * HCAP, L),
        rj.reshape(NW * HCAP, L),
        ii3, ij3, bias1d)
    return pi, pj, pw


def kernel(user, item_i, item_j, embed_user, embed_item, item_biases):
    u = user.astype(jnp.int32)
    ii = item_i.astype(jnp.int32)
    ij = item_j.astype(jnp.int32)
    pi, pj, pw_t = _bpr_sc(
        u, ii, ij, embed_user.T.reshape(2, 8, USER_NUM),
        embed_item.T.reshape(2, 8, ITEM_NUM),
        item_biases.T.reshape(ITEM_NUM),
        ii.reshape(NW, NCHUNK, CHUNK), ij.reshape(NW, NCHUNK, CHUNK))
    return pi.reshape(BATCH), pj.reshape(BATCH), pw_t.T


# speedup vs baseline: 1.1158x; 1.0170x over previous
import functools

import jax
import jax.numpy as jnp
from jax import lax
from jax.experimental import pallas as pl
from jax.experimental.pallas import tpu as pltpu
from jax.experimental.pallas import tpu_sc as plsc

USER_NUM = 1000000
ITEM_NUM = 1000000
FACTOR = 16
BATCH = 16384

L = 16
NW = 32
BPW = BATCH // NW
NCHUNK = 4
CHUNK = BPW // NCHUNK

CHW = 512
NFULL = USER_NUM // CHW
CH_BIG = NFULL - 61 * NW
LASTC0 = (NFULL - 1) * CHW
HCAP = 1024
HSLACK = HCAP + L
SENT = BATCH + 64
INVN = BATCH + BPW


def _p1_body(user_h, itemi_h, itemj_h, eu_t, ei_t,
             rows_u_o, rows_i_o, rows_j_o, inv_u_o, inv_i_o, inv_j_o,
             slab_u, slab_v, slab_t, slab_t2, idxbuf,
             hidx_u, hpos_u, hidx_i, hpos_i, hidx_j, hpos_j,
             rows_u, rows_i, rows_j,
             slots_u, posb_u, slots_i, posb_i, slots_j, posb_j,
             cidx, cpos, sem):
    wid = lax.axis_index("s") * 2 + lax.axis_index("c")
    start = jnp.where(wid < CH_BIG, 62 * wid, 61 * wid + CH_BIG)
    rlo = start * CHW
    rhi = jnp.where(wid == NW - 1, USER_NUM, (start + 61) * CHW
                    + jnp.where(wid < CH_BIG, CHW, 0))

    iota = lax.iota(jnp.int32, L)

    sentv = jnp.full((L,), SENT, jnp.int32)
    for posb in (posb_u, posb_i, posb_j):
        for g in range(8):
            for v in range(128 // L):
                plsc.store_scatter(
                    posb, [jnp.full((L,), g, jnp.int32), v * L + iota], sentv)

    def bin_set(idx_h, hidx, hpos):
        cur = jnp.int32(0)
        for p in range(BATCH // 2048):
            pltpu.sync_copy(idx_h.at[pl.ds(p * 2048, 2048)], idxbuf)

            def scan(i, c):
                x = idxbuf[pl.ds(i * L, L)]
                m = (x >= rlo) & (x < rhi)
                n = plsc.all_reduce_population_count(m)[0]

                @pl.when(n > 0)
                def _():
                    pos = p * 2048 + i * L + iota
                    plsc.store_compressed(hidx.at[pl.ds(c, L)], x, mask=m)
                    plsc.store_compressed(hpos.at[pl.ds(c, L)], pos, mask=m)

                return c + n

            cur = lax.fori_loop(0, 2048 // L, scan, cur)
        return cur

    nu = bin_set(user_h, hidx_u, hpos_u)
    ni = bin_set(itemi_h, hidx_i, hpos_i)
    nj = bin_set(itemj_h, hidx_j, hpos_j)

    def serve(slab, hidx, hpos, nh, rows, slots, posb, clo, chi, c0, gc):
        def cscan(v, cc):
            x = hidx[pl.ds(v * L, L)]
            pp = hpos[pl.ds(v * L, L)]
            m = (x >= clo) & (x < chi) & ((v * L + iota) < nh)
            n = plsc.all_reduce_population_count(m)[0]

            @pl.when(n > 0)
            def _():
                plsc.store_compressed(cidx.at[pl.ds(cc, L)], x - c0, mask=m)
                plsc.store_compressed(cpos.at[pl.ds(cc, L)], pp, mask=m)

            return cc + n

        ccnt = lax.fori_loop(0, (nh + L - 1) // L, cscan, jnp.int32(0))

        def emitg(gi, carry):
            lane = gi * L
            valid = (lane + iota) < ccnt
            cols = jnp.where(valid, cidx[pl.ds(lane, L)], 0)
            poss = cpos[pl.ds(lane, L)]
            slotv = gc + lane + iota
            ct = lax.shift_right_logical(cols, 7)
            cl = cols & 127
            for f in range(FACTOR):
                hv = jnp.full((L,), f // 8, jnp.int32)
                fv = jnp.full((L,), f % 8, jnp.int32)
                vals = plsc.load_gather(slab, [hv, ct, fv, cl])
                plsc.store_scatter(rows, [slotv * L + f], vals, mask=valid)
            rowp = lax.shift_right_logical(slotv, 7)
            colp = slotv & 127
            plsc.store_scatter(posb, [rowp, colp], poss, mask=valid)
            plsc.store_scatter(slots, [rowp, colp], wid * HCAP + slotv,
                               mask=valid)
            return carry

        lax.fori_loop(0, (ccnt + L - 1) // L, emitg, jnp.int32(0))
        return gc + ccnt

    def chunk_step(k, carry):
        gu, gi, gj = carry
        ch = start + k
        clo = ch * CHW
        chi = jnp.minimum(clo + CHW, NFULL * CHW)
        c0 = pl.multiple_of(jnp.minimum(clo, LASTC0), CHW)
        cps = []
        for h in range(2):
            for t in range(CHW // 128):
                src = pl.ds(c0 + t * 128, 128)
                cps.append(pltpu.async_copy(
                    eu_t.at[h, :, src], slab_u.at[h, t], sem))
                cps.append(pltpu.async_copy(
                    ei_t.at[h, :, src], slab_v.at[h, t], sem))
        for cp in cps:
            cp.wait()
        gu2 = serve(slab_u, hidx_u, hpos_u, nu, rows_u, slots_u, posb_u,
                    clo, chi, c0, gu)
        gi2 = serve(slab_v, hidx_i, hpos_i, ni, rows_i, slots_i, posb_i,
                    clo, chi, c0, gi)
        gj2 = serve(slab_v, hidx_j, hpos_j, nj, rows_j, slots_j, posb_j,
                    clo, chi, c0, gj)
        return gu2, gi2, gj2

    gu, gi, gj = lax.fori_loop(
        0, 62, chunk_step, (jnp.int32(0), jnp.int32(0), jnp.int32(0)))

    @pl.when(wid == NW - 1)
    def _():
        tl = jnp.int32(NFULL * CHW)
        cps = []
        for h in range(2):
            cps.append(pltpu.async_copy(
                eu_t.at[h, :, pl.ds(NFULL * CHW, 64)], slab_t.at[h, 0], sem))
            cps.append(pltpu.async_copy(
                ei_t.at[h, :, pl.ds(NFULL * CHW, 64)], slab_t2.at[h, 0], sem))
        for cp in cps:
            cp.wait()
        serve(slab_t, hidx_u, hpos_u, nu, rows_u, slots_u, posb_u,
              tl, jnp.int32(USER_NUM), tl, gu)
        serve(slab_t2, hidx_i, hpos_i, ni, rows_i, slots_i, posb_i,
              tl, jnp.int32(USER_NUM), tl, gi)
        serve(slab_t2, hidx_j, hpos_j, nj, rows_j, slots_j, posb_j,
              tl, jnp.int32(USER_NUM), tl, gj)

    base = wid * HCAP * L
    pltpu.sync_copy(rows_u, rows_u_o.at[pl.ds(base, HCAP * L)])
    pltpu.sync_copy(rows_i, rows_i_o.at[pl.ds(base, HCAP * L)])
    pltpu.sync_copy(rows_j, rows_j_o.at[pl.ds(base, HCAP * L)])
    for inv_o, slots, posb in ((inv_u_o, slots_u, posb_u),
                               (inv_i_o, slots_i, posb_i),
                               (inv_j_o, slots_j, posb_j)):
        for g in range(8):
            pltpu.async_copy(slots.at[g], inv_o.at[posb.at[g]], sem).wait()


def _p2_body(inv_u_h, inv_i_h, inv_j_h, ru_h, ri_h, rj_h,
             item_i_h, item_j_h, bias_h,
             out_pi, out_pj, out_pw,
             inv_u, inv_i, inv_j, idx_i, idx_j,
             rows_u, rows_i, rows_j, bias_i_v, bias_j_v,
             pw_v, pred_i_v, pred_j_v, sem):
    wid = lax.axis_index("s") * 2 + lax.axis_index("c")
    base = wid * BPW

    pltpu.sync_copy(inv_u_h.at[wid], inv_u)
    pltpu.sync_copy(inv_i_h.at[wid], inv_i)
    pltpu.sync_copy(inv_j_h.at[wid], inv_j)
    pltpu.sync_copy(item_i_h.at[wid], idx_i)
    pltpu.sync_copy(item_j_h.at[wid], idx_j)

    copies = []
    for c in range(NCHUNK):
        dst = pl.ds(c * CHUNK, CHUNK)
        copies.append(pltpu.async_copy(ru_h.at[inv_u.at[c]], rows_u.at[dst], sem))
        copies.append(pltpu.async_copy(ri_h.at[inv_i.at[c]], rows_i.at[dst], sem))
        copies.append(pltpu.async_copy(rj_h.at[inv_j.at[c]], rows_j.at[dst], sem))
        copies.append(pltpu.async_copy(bias_h.at[idx_i.at[c]], bias_i_v.at[dst], sem))
        copies.append(pltpu.async_copy(bias_h.at[idx_j.at[c]], bias_j_v.at[dst], sem))
    for cp in copies:
        cp.wait()

    iota = lax.iota(jnp.int32, L)

    def block(b, carry):
        rbase = b * L
        ridx = rbase + iota
        acc_i = bias_i_v[pl.ds(rbase, L)]
        acc_j = bias_j_v[pl.ds(rbase, L)]
        for f in range(FACTOR):
            cf = jnp.full((L,), f, jnp.int32)
            uc = plsc.load_gather(rows_u, [ridx, cf])
            ic = plsc.load_gather(rows_i, [ridx, cf])
            jc = plsc.load_gather(rows_j, [ridx, cf])
            pwc = uc * ic
            plsc.store_scatter(pw_v, [cf, ridx], pwc)
            acc_i = acc_i + pwc
            acc_j = acc_j + uc * jc
        pred_i_v[pl.ds(rbase, L)] = acc_i
        pred_j_v[pl.ds(rbase, L)] = acc_j
        return carry

    lax.fori_loop(0, BPW // L, block, 0)

    pltpu.sync_copy(pred_i_v, out_pi.at[wid])
    pltpu.sync_copy(pred_j_v, out_pj.at[wid])
    pltpu.sync_copy(pw_v, out_pw.at[:, pl.ds(base, BPW)])


@jax.jit
def _bpr_sc(user, item_i, item_j, eu_t, ei_t, bias1d, ii3, ij3):
    mesh = plsc.VectorSubcoreMesh(core_axis_name="c", subcore_axis_name="s")
    p1 = functools.partial(
        pl.kernel,
        mesh=mesh,
        compiler_params=pltpu.CompilerParams(
            needs_layout_passes=False, use_tc_tiling_on_sc=True),
        out_type=[
            jax.ShapeDtypeStruct((NW * HCAP * L,), jnp.float32),
            jax.ShapeDtypeStruct((NW * HCAP * L,), jnp.float32),
            jax.ShapeDtypeStruct((NW * HCAP * L,), jnp.float32),
            jax.ShapeDtypeStruct((INVN,), jnp.int32),
            jax.ShapeDtypeStruct((INVN,), jnp.int32),
            jax.ShapeDtypeStruct((INVN,), jnp.int32),
        ],
        scratch_types=[
            pltpu.VMEM((2, CHW // 128, 8, 128), jnp.float32),
            pltpu.VMEM((2, CHW // 128, 8, 128), jnp.float32),
            pltpu.VMEM((2, 1, 8, 64), jnp.float32),
            pltpu.VMEM((2, 1, 8, 64), jnp.float32),
            pltpu.VMEM((2048,), jnp.int32),
            pltpu.VMEM((HSLACK,), jnp.int32),
            pltpu.VMEM((HSLACK,), jnp.int32),
            pltpu.VMEM((HSLACK,), jnp.int32),
            pltpu.VMEM((HSLACK,), jnp.int32),
            pltpu.VMEM((HSLACK,), jnp.int32),
            pltpu.VMEM((HSLACK,), jnp.int32),
            pltpu.VMEM((HCAP * L,), jnp.float32),
            pltpu.VMEM((HCAP * L,), jnp.float32),
            pltpu.VMEM((HCAP * L,), jnp.float32),
            pltpu.VMEM((8, 128), jnp.int32),
            pltpu.VMEM((8, 128), jnp.int32),
            pltpu.VMEM((8, 128), jnp.int32),
            pltpu.VMEM((8, 128), jnp.int32),
            pltpu.VMEM((8, 128), jnp.int32),
            pltpu.VMEM((8, 128), jnp.int32),
            pltpu.VMEM((112,), jnp.int32),
            pltpu.VMEM((112,), jnp.int32),
            pltpu.SemaphoreType.DMA,
        ],
    )(_p1_body)
    ru, ri, rj, ivu, ivi, ivj = p1(user, item_i, item_j, eu_t, ei_t)

    p2 = functools.partial(
        pl.kernel,
        mesh=mesh,
        compiler_params=pltpu.CompilerParams(
            needs_layout_passes=False, use_tc_tiling_on_sc=False),
        out_type=[
            jax.ShapeDtypeStruct((NW, BPW), jnp.float32),
            jax.ShapeDtypeStruct((NW, BPW), jnp.float32),
            jax.ShapeDtypeStruct((FACTOR, BATCH), jnp.float32),
        ],
        scratch_types=[
            pltpu.VMEM((NCHUNK, CHUNK), jnp.int32),
            pltpu.VMEM((NCHUNK, CHUNK), jnp.int32),
            pltpu.VMEM((NCHUNK, CHUNK), jnp.int32),
            pltpu.VMEM((NCHUNK, CHUNK), jnp.int32),
            pltpu.VMEM((NCHUNK, CHUNK), jnp.int32),
            pltpu.VMEM((BPW, FACTOR), jnp.float32),
            pltpu.VMEM((BPW, FACTOR), jnp.float32),
            pltpu.VMEM((BPW, FACTOR), jnp.float32),
            pltpu.VMEM((BPW,), jnp.float32),
            pltpu.VMEM((BPW,), jnp.float32),
            pltpu.VMEM((FACTOR, BPW), jnp.float32),
            pltpu.VMEM((BPW,), jnp.float32),
            pltpu.VMEM((BPW,), jnp.float32),
            pltpu.SemaphoreType.DMA,
        ],
    )(_p2_body)
    pi, pj, pw = p2(
        ivu.reshape(INVN // BPW, NCHUNK, CHUNK),
        ivi.reshape(INVN // BPW, NCHUNK, CHUNK),
        ivj.reshape(INVN // BPW, NCHUNK, CHUNK),
        ru.reshape(NW * HCAP, L), ri.reshape(NW * HCAP, L),
        rj.reshape(NW * HCAP, L),
        ii3, ij3, bias1d)
    return pi, pj, pw


def kernel(user, item_i, item_j, embed_user, embed_item, item_biases):
    u = user.astype(jnp.int32)
    ii = item_i.astype(jnp.int32)
    ij = item_j.astype(jnp.int32)
    pi, pj, pw_t = _bpr_sc(
        u, ii, ij, embed_user.T.reshape(2, 8, USER_NUM),
        embed_item.T.reshape(2, 8, ITEM_NUM),
        item_biases.T.reshape(ITEM_NUM),
        ii.reshape(NW, NCHUNK, CHUNK), ij.reshape(NW, NCHUNK, CHUNK))
    return pi.reshape(BATCH), pj.reshape(BATCH), pw_t.T
